# baseline (speedup 1.0000x reference)
import functools

import jax
import jax.numpy as jnp
import numpy as _np
from jax import lax
from jax.experimental import pallas as pl
from jax.experimental.pallas import tpu as pltpu
from jax.experimental.pallas import tpu_sc as plsc

B = 16384
EMB = 32
NB = 1000
VOCAB = 100000
NROW = VOCAB + 1
OUT_W = 2 * EMB + 1

NC = 2
NS = 16
L = 16
NW = NC * NS
BPW = B // NW

_INV_STD = float(1.0 / _np.sqrt(_np.float32(1.0 / 12.0)))

_mesh = plsc.VectorSubcoreMesh(
    core_axis_name="c", subcore_axis_name="s", num_cores=NC, num_subcores=NS
)

_params = pltpu.CompilerParams(
    needs_layout_passes=False, use_tc_tiling_on_sc=False
)


@functools.partial(
    pl.kernel,
    out_type=jax.ShapeDtypeStruct((EMB + 1, B), jnp.float32),
    mesh=_mesh,
    compiler_params=_params,
    scratch_types=[
        pltpu.VMEM((BPW,), jnp.float32),
        pltpu.VMEM((BPW,), jnp.int32),
        pltpu.VMEM((BPW,), jnp.float32),
        pltpu.VMEM((BPW, EMB), jnp.float32),
        pltpu.VMEM((EMB, BPW), jnp.float32),
        pltpu.VMEM((NB,), jnp.float32),
        pltpu.SemaphoreType.DMA,
        pltpu.SemaphoreType.DMA,
        pltpu.SemaphoreType.DMA,
    ],
)
def _ts_side_sc(
    ts_hbm, tt_hbm, bk_hbm, out_hbm,
    ts_v, bidx_v, nrm_v, te_v, tet_v, bk_v,
    sem_te, sem_o2, sem_o3,
):
    wid = lax.axis_index("s") * NC + lax.axis_index("c")
    base = wid * BPW

    pltpu.sync_copy(ts_hbm.at[pl.ds(base, BPW)], ts_v)
    pltpu.sync_copy(bk_hbm, bk_v)

    iota = lax.iota(jnp.int32, L)

    @plsc.parallel_loop(0, BPW // L)
    def _search(g):
        off = g * L
        t = ts_v[pl.ds(off, L)]
        lo = jnp.zeros((L,), jnp.int32)
        hi = jnp.full((L,), NB, jnp.int32)
        for _ in range(10):
            mid = lax.shift_right_logical(lo + hi, 1)
            bv = plsc.load_gather(bk_v, [mid])
            le = bv <= t
            lo = jnp.where(le, mid + 1, lo)
            hi = jnp.where(le, hi, mid)
        bidx_v[pl.ds(off, L)] = lo
        nrm_v[pl.ds(off, L)] = (t - 0.5) * _INV_STD

    te_cp = pltpu.async_copy(tt_hbm.at[bidx_v], te_v, sem_te)
    o3 = pltpu.async_copy(
        nrm_v, out_hbm.at[EMB, pl.ds(base, BPW)], sem_o3
    )
    te_cp.wait()

    @plsc.parallel_loop(0, BPW // L)
    def _tr_te(g):
        off = g * L
        rows = off + iota
        for c in range(EMB):
            tet_v[c, pl.ds(off, L)] = plsc.load_gather(
                te_v, [rows, jnp.full((L,), c, jnp.int32)]
            )

    o2 = pltpu.async_copy(
        tet_v, out_hbm.at[pl.ds(0, EMB), pl.ds(base, BPW)], sem_o2
    )
    o2.wait()
    o3.wait()


@functools.partial(
    pl.kernel,
    out_type=jax.ShapeDtypeStruct((EMB, B), jnp.float32),
    mesh=_mesh,
    compiler_params=_params,
    scratch_types=[
        pltpu.VMEM((BPW,), jnp.int32),
        pltpu.VMEM((BPW * EMB,), jnp.int32),
        pltpu.VMEM((BPW * EMB,), jnp.float32),
        pltpu.VMEM((EMB, BPW), jnp.float32),
        pltpu.SemaphoreType.DMA,
        pltpu.SemaphoreType.DMA,
    ],
)
def _user_side_sc(
    uid_hbm, utf_hbm, out_hbm,
    idx_v, gix_v, gbuf_v, uet_v,
    sem_ue, sem_o1,
):
    wid = lax.axis_index("s") * NC + lax.axis_index("c")
    base = wid * BPW

    pltpu.sync_copy(uid_hbm.at[pl.ds(base, BPW)], idx_v)

    @plsc.parallel_loop(0, BPW // L)
    def _mkidx(g):
        off = g * L
        uvec = idx_v[pl.ds(off, L)]
        for c in range(EMB):
            gix_v[pl.ds(c * BPW + off, L)] = uvec + c * NROW

    ue_cp = pltpu.async_copy(utf_hbm.at[gix_v], gbuf_v, sem_ue)
    ue_cp.wait()

    @plsc.parallel_loop(0, BPW * EMB // (L * L))
    def _bridge(k):
        c = k // 2
        half = (k % 2) * (BPW // 2)
        for j in range(L):
            off = half + j * L
            uet_v[c, pl.ds(off, L)] = gbuf_v[pl.ds(c * BPW + off, L)]

    o1 = pltpu.async_copy(
        uet_v, out_hbm.at[pl.ds(0, EMB), pl.ds(base, BPW)], sem_o1
    )
    o1.wait()


def kernel(user_id, timestamp, user_table, ts_table, buckets):
    uid = user_id.astype(jnp.int32)
    utf = user_table.T.reshape(-1)
    te_norm_t = _ts_side_sc(timestamp, ts_table, buckets)
    ue_t = _user_side_sc(uid, utf)
    out_t = jnp.concatenate([ue_t, te_norm_t], axis=0)
    return out_t.T


# speedup vs baseline: 1.2846x; 1.2846x over previous
import functools

import jax
import jax.numpy as jnp
import numpy as _np
from jax import lax
from jax.experimental import pallas as pl
from jax.experimental.pallas import tpu as pltpu
from jax.experimental.pallas import tpu_sc as plsc

B = 16384
EMB = 32
NB = 1000
VOCAB = 100000
NROW = VOCAB + 1
OUT_W = 2 * EMB + 1

NC = 2
NS = 16
L = 16
NW = NC * NS
BPW = B // NW

_INV_STD = float(1.0 / _np.sqrt(_np.float32(1.0 / 12.0)))

_mesh = plsc.VectorSubcoreMesh(
    core_axis_name="c", subcore_axis_name="s", num_cores=NC, num_subcores=NS
)


@functools.partial(
    pl.kernel,
    out_type=jax.ShapeDtypeStruct((OUT_W, B), jnp.float32),
    mesh=_mesh,
    compiler_params=pltpu.CompilerParams(
        needs_layout_passes=False, use_tc_tiling_on_sc=False
    ),
    scratch_types=[
        pltpu.VMEM((BPW,), jnp.int32),
        pltpu.VMEM((BPW,), jnp.float32),
        pltpu.VMEM((BPW,), jnp.int32),
        pltpu.VMEM((BPW,), jnp.float32),
        pltpu.VMEM((BPW * EMB,), jnp.int32),
        pltpu.VMEM((BPW * EMB,), jnp.float32),
        pltpu.VMEM((EMB, BPW), jnp.float32),
        pltpu.VMEM((BPW, EMB), jnp.float32),
        pltpu.VMEM((EMB, BPW), jnp.float32),
        pltpu.VMEM((NB,), jnp.float32),
        pltpu.SemaphoreType.DMA,
        pltpu.SemaphoreType.DMA,
        pltpu.SemaphoreType.DMA,
        pltpu.SemaphoreType.DMA,
        pltpu.SemaphoreType.DMA,
    ],
)
def _user_model_sc(
    uid_hbm, ts_hbm, utf_hbm, tt_hbm, bk_hbm, out_hbm,
    idx_v, ts_v, bidx_v, nrm_v, gix_v, gbuf_v, uet_v, te_v, tet_v, bk_v,
    sem_ue, sem_te, sem_o1, sem_o2, sem_o3,
):
    wid = lax.axis_index("s") * NC + lax.axis_index("c")
    base = wid * BPW

    pltpu.sync_copy(uid_hbm.at[pl.ds(base, BPW)], idx_v)

    iota = lax.iota(jnp.int32, L)

    @plsc.parallel_loop(0, BPW // L)
    def _mkidx(g):
        off = g * L
        uvec = idx_v[pl.ds(off, L)]
        for c in range(EMB):
            gix_v[pl.ds(c * BPW + off, L)] = uvec + c * NROW

    ue_cp = pltpu.async_copy(utf_hbm.at[gix_v], gbuf_v, sem_ue)

    pltpu.sync_copy(ts_hbm.at[pl.ds(base, BPW)], ts_v)
    pltpu.sync_copy(bk_hbm, bk_v)

    @plsc.parallel_loop(0, BPW // L)
    def _search(g):
        off = g * L
        t = ts_v[pl.ds(off, L)]
        lo = jnp.zeros((L,), jnp.int32)
        hi = jnp.full((L,), NB, jnp.int32)
        for _ in range(10):
            mid = lax.shift_right_logical(lo + hi, 1)
            bv = plsc.load_gather(bk_v, [mid])
            le = bv <= t
            lo = jnp.where(le, mid + 1, lo)
            hi = jnp.where(le, hi, mid)
        bidx_v[pl.ds(off, L)] = lo
        nrm_v[pl.ds(off, L)] = (t - 0.5) * _INV_STD

    te_cp = pltpu.async_copy(tt_hbm.at[bidx_v], te_v, sem_te)
    o3 = pltpu.async_copy(
        nrm_v, out_hbm.at[2 * EMB, pl.ds(base, BPW)], sem_o3
    )
    te_cp.wait()

    @plsc.parallel_loop(0, BPW // L)
    def _tr_te(g):
        off = g * L
        rows = off + iota
        for c in range(EMB):
            tet_v[c, pl.ds(off, L)] = plsc.load_gather(
                te_v, [rows, jnp.full((L,), c, jnp.int32)]
            )

    o2 = pltpu.async_copy(
        tet_v, out_hbm.at[pl.ds(EMB, EMB), pl.ds(base, BPW)], sem_o2
    )

    ue_cp.wait()

    @plsc.parallel_loop(0, BPW * EMB // (L * L))
    def _bridge(k):
        c = k // 2
        half = (k % 2) * (BPW // 2)
        for j in range(L):
            off = half + j * L
            uet_v[c, pl.ds(off, L)] = gbuf_v[pl.ds(c * BPW + off, L)]

    o1 = pltpu.async_copy(
        uet_v, out_hbm.at[pl.ds(0, EMB), pl.ds(base, BPW)], sem_o1
    )
    o1.wait()
    o2.wait()
    o3.wait()


def kernel(user_id, timestamp, user_table, ts_table, buckets):
    uid = user_id.astype(jnp.int32)
    utf = user_table.T.reshape(-1)
    out_t = _user_model_sc(uid, timestamp, utf, ts_table, buckets)
    return out_t.T
